# baseline (speedup 1.0000x reference)
import functools
import math

import jax
import jax.numpy as jnp
from jax import lax
from jax.experimental import pallas as pl
from jax.experimental.pallas import tpu as pltpu
from jax.experimental.pallas import tpu_sc as plsc

_NUM_NBS = 12
_R0 = 0.5
_D0 = 0.3
_N = 4096
_NC = 2
_NS = 16
_NW = _NC * _NS
_APW = _N // _NW
_NG = _APW // 16


def _sh6_constants():
    l = 6
    n = []
    for m in range(0, l + 1):
        nlm = math.sqrt(
            (2 * l + 1) / (4.0 * math.pi)
            * math.factorial(l - m) / math.factorial(l + m))
        n.append(nlm if m == 0 else math.sqrt(2.0) * nlm)
    return n


_C = _sh6_constants()


def _rsqrt_nr(x):
    i = plsc.bitcast(x, jnp.int32)
    i = jnp.int32(0x5F3759DF) - lax.shift_right_logical(i, 1)
    y = plsc.bitcast(i, jnp.float32)
    for _ in range(3):
        y = y * (1.5 - 0.5 * x * y * y)
    return y


def _sc_body(x_hbm, y_hbm, z_hbm, out_hbm, x_v, y_v, z_v, nbr_v, qn_v):
    wid = lax.axis_index("s") * _NC + lax.axis_index("c")
    base = wid * _APW
    lane = jnp.arange(16, dtype=jnp.int32)

    pltpu.sync_copy(x_hbm, x_v)
    pltpu.sync_copy(y_hbm, y_v)
    pltpu.sync_copy(z_hbm, z_v)

    inf = jnp.array(jnp.inf, jnp.float32)

    def group_body(g, total):
        gbase = base + g * 16

        def atom_body(aa, _):
            gi = gbase + aa
            gi_v = jnp.full((16,), 0, jnp.int32) + gi
            ax = plsc.load_gather(x_v, [gi_v])
            ay = plsc.load_gather(y_v, [gi_v])
            az = plsc.load_gather(z_v, [gi_v])

            def chunk_body(c, carry):
                rk, rv, rmax = carry
                sl = pl.ds(c * 16, 16)
                dx = x_v[sl] - ax
                dy = y_v[sl] - ay
                dz = z_v[sl] - az
                d2 = dx * dx + dy * dy + dz * dz
                vals = lane + c * 16
                d2 = jnp.where(vals == gi_v, inf, d2)

                def merge(_):
                    ck, cv = plsc.sort_key_val(d2, vals)
                    rck = lax.rev(ck, (0,))
                    rcv = lax.rev(cv, (0,))
                    sel = rk <= rck
                    mk = jnp.where(sel, rk, rck)
                    mv = jnp.where(sel, rv, rcv)
                    nk, nv = plsc.sort_key_val(mk, mv)
                    return nk, nv, jnp.max(nk)

                def keep(_):
                    return rk, rv, rmax

                return lax.cond(jnp.min(d2) < rmax, merge, keep, 0)

            rk0 = jnp.full((16,), jnp.inf, jnp.float32)
            rv0 = jnp.zeros((16,), jnp.int32)
            _, rv, _ = lax.fori_loop(0, _N // 16, chunk_body, (rk0, rv0, inf))
            plsc.store_scatter(nbr_v, [aa * 16 + lane], rv)
            return 0

        lax.fori_loop(0, 16, atom_body, 0)

        ax16 = x_v[pl.ds(gbase, 16)]
        ay16 = y_v[pl.ds(gbase, 16)]
        az16 = z_v[pl.ds(gbase, 16)]
        zero = jnp.zeros((16,), jnp.float32)
        acc = [zero] * 13
        ssum = zero
        for k in range(_NUM_NBS):
            nidx = plsc.load_gather(nbr_v, [lane * 16 + k])
            dx = plsc.load_gather(x_v, [nidx]) - ax16
            dy = plsc.load_gather(y_v, [nidx]) - ay16
            dz = plsc.load_gather(z_v, [nidx]) - az16
            d2 = dx * dx + dy * dy + dz * dz
            r = _rsqrt_nr(d2)
            dd = d2 * r
            ux = dx * r
            uy = dy * r
            uz = dz * r
            t = dd - _D0
            sig = jnp.exp(t * t * (-1.0 / (2.0 * _R0 * _R0)))
            ssum = ssum + sig
            z2 = uz * uz
            p = [
                (((231.0 * z2 - 315.0) * z2 + 105.0) * z2 - 5.0) * (1.0 / 16.0),
                ((1386.0 * z2 - 1260.0) * z2 + 210.0) * uz * (1.0 / 16.0),
                ((6930.0 * z2 - 3780.0) * z2 + 210.0) * (1.0 / 16.0),
                (27720.0 * z2 - 7560.0) * uz * (1.0 / 16.0),
                (83160.0 * z2 - 7560.0) * (1.0 / 16.0),
                10395.0 * uz,
            ]
            acc[0] = acc[0] + sig * p[0]
            a = ux
            b = uy
            for m in range(1, 7):
                if m < 6:
                    acc[2 * m - 1] = acc[2 * m - 1] + sig * (p[m] * a)
                    acc[2 * m] = acc[2 * m] + sig * (p[m] * b)
                    a, b = a * ux - b * uy, a * uy + b * ux
                else:
                    acc[11] = acc[11] + sig * a
                    acc[12] = acc[12] + sig * b
        norm2 = jnp.zeros((16,), jnp.float32)
        cs = [_C[0]] + [c for m in range(1, 7) for c in
                        ((_C[m], _C[m]) if m < 6 else
                         (_C[6] * 10395.0, _C[6] * 10395.0))]
        for m in range(13):
            s = cs[m] * acc[m]
            norm2 = norm2 + s * s
        rn = _rsqrt_nr(norm2)
        qn = norm2 * rn / ssum
        return total + qn

    total = lax.fori_loop(0, _NG, group_body, jnp.zeros((16,), jnp.float32))
    qn_v[...] = total * (1.0 / _N)
    pltpu.sync_copy(qn_v, out_hbm.at[wid])


def kernel(positions):
    pos = positions.astype(jnp.float32)
    mesh = plsc.VectorSubcoreMesh(
        core_axis_name="c", subcore_axis_name="s",
        num_cores=_NC, num_subcores=_NS)
    f = functools.partial(
        pl.kernel,
        out_type=jax.ShapeDtypeStruct((_NW, 16), jnp.float32),
        mesh=mesh,
        compiler_params=pltpu.CompilerParams(needs_layout_passes=False),
        scratch_types=[
            pltpu.VMEM((_N,), jnp.float32),
            pltpu.VMEM((_N,), jnp.float32),
            pltpu.VMEM((_N,), jnp.float32),
            pltpu.VMEM((256,), jnp.int32),
            pltpu.VMEM((16,), jnp.float32),
        ],
    )(_sc_body)
    out = f(pos[:, 0], pos[:, 1], pos[:, 2])
    return jnp.sum(out)


# speedup vs baseline: 2.2774x; 2.2774x over previous
import functools
import math

import jax
import jax.numpy as jnp
from jax import lax
from jax.experimental import pallas as pl
from jax.experimental.pallas import tpu as pltpu
from jax.experimental.pallas import tpu_sc as plsc

_NUM_NBS = 12
_R0 = 0.5
_D0 = 0.3
_N = 4096

_TC_N = 2560
_SC_N = _N - _TC_N

_NC = 2
_NS = 16
_NW = _NC * _NS
_APW = _SC_N // _NW
_NG = _APW // 16

_ROWS = 256


def _sh6_constants():
    l = 6
    n = []
    for m in range(0, l + 1):
        nlm = math.sqrt(
            (2 * l + 1) / (4.0 * math.pi)
            * math.factorial(l - m) / math.factorial(l + m))
        n.append(nlm if m == 0 else math.sqrt(2.0) * nlm)
    return n


_C = _sh6_constants()


def _rsqrt_nr(x):
    i = plsc.bitcast(x, jnp.int32)
    i = jnp.int32(0x5F3759DF) - lax.shift_right_logical(i, 1)
    y = plsc.bitcast(i, jnp.float32)
    for _ in range(3):
        y = y * (1.5 - 0.5 * x * y * y)
    return y


def _sc_body(x_hbm, y_hbm, z_hbm, out_hbm, x_v, y_v, z_v, nbr_v, qn_v):
    wid = lax.axis_index("s") * _NC + lax.axis_index("c")
    base = _TC_N + wid * _APW
    lane = jnp.arange(16, dtype=jnp.int32)

    pltpu.sync_copy(x_hbm, x_v)
    pltpu.sync_copy(y_hbm, y_v)
    pltpu.sync_copy(z_hbm, z_v)

    inf = jnp.array(jnp.inf, jnp.float32)

    def group_body(g, total):
        gbase = base + g * 16

        def atom_body(aa, _):
            gi = gbase + aa
            gi_v = jnp.full((16,), 0, jnp.int32) + gi
            ax = plsc.load_gather(x_v, [gi_v])
            ay = plsc.load_gather(y_v, [gi_v])
            az = plsc.load_gather(z_v, [gi_v])

            def chunk_body(c, carry):
                rk, rv, rmax = carry
                sl = pl.ds(c * 16, 16)
                dx = x_v[sl] - ax
                dy = y_v[sl] - ay
                dz = z_v[sl] - az
                d2 = dx * dx + dy * dy + dz * dz
                vals = lane + c * 16
                d2 = jnp.where(vals == gi_v, inf, d2)

                def merge(_):
                    ck, cv = plsc.sort_key_val(d2, vals)
                    rck = lax.rev(ck, (0,))
                    rcv = lax.rev(cv, (0,))
                    sel = rk <= rck
                    mk = jnp.where(sel, rk, rck)
                    mv = jnp.where(sel, rv, rcv)
                    nk, nv = plsc.sort_key_val(mk, mv)
                    return nk, nv, jnp.max(nk)

                def keep(_):
                    return rk, rv, rmax

                return lax.cond(jnp.min(d2) < rmax, merge, keep, 0)

            rk0 = jnp.full((16,), jnp.inf, jnp.float32)
            rv0 = jnp.zeros((16,), jnp.int32)
            _, rv, _ = lax.fori_loop(0, _N // 16, chunk_body, (rk0, rv0, inf))
            plsc.store_scatter(nbr_v, [aa * 16 + lane], rv)
            return 0

        lax.fori_loop(0, 16, atom_body, 0)

        ax16 = x_v[pl.ds(gbase, 16)]
        ay16 = y_v[pl.ds(gbase, 16)]
        az16 = z_v[pl.ds(gbase, 16)]
        zero = jnp.zeros((16,), jnp.float32)
        acc = [zero] * 13
        ssum = zero
        for k in range(_NUM_NBS):
            nidx = plsc.load_gather(nbr_v, [lane * 16 + k])
            dx = plsc.load_gather(x_v, [nidx]) - ax16
            dy = plsc.load_gather(y_v, [nidx]) - ay16
            dz = plsc.load_gather(z_v, [nidx]) - az16
            d2 = dx * dx + dy * dy + dz * dz
            r = _rsqrt_nr(d2)
            dd = d2 * r
            ux = dx * r
            uy = dy * r
            uz = dz * r
            t = dd - _D0
            sig = jnp.exp(t * t * (-1.0 / (2.0 * _R0 * _R0)))
            ssum = ssum + sig
            z2 = uz * uz
            p = [
                (((231.0 * z2 - 315.0) * z2 + 105.0) * z2 - 5.0) * (1.0 / 16.0),
                ((1386.0 * z2 - 1260.0) * z2 + 210.0) * uz * (1.0 / 16.0),
                ((6930.0 * z2 - 3780.0) * z2 + 210.0) * (1.0 / 16.0),
                (27720.0 * z2 - 7560.0) * uz * (1.0 / 16.0),
                (83160.0 * z2 - 7560.0) * (1.0 / 16.0),
                10395.0 * uz,
            ]
            acc[0] = acc[0] + sig * p[0]
            a = ux
            b = uy
            for m in range(1, 7):
                if m < 6:
                    acc[2 * m - 1] = acc[2 * m - 1] + sig * (p[m] * a)
                    acc[2 * m] = acc[2 * m] + sig * (p[m] * b)
                    a, b = a * ux - b * uy, a * uy + b * ux
                else:
                    acc[11] = acc[11] + sig * a
                    acc[12] = acc[12] + sig * b
        norm2 = jnp.zeros((16,), jnp.float32)
        cs = [_C[0]] + [c for m in range(1, 7) for c in
                        ((_C[m], _C[m]) if m < 6 else
                         (_C[6] * 10395.0, _C[6] * 10395.0))]
        for m in range(13):
            s = cs[m] * acc[m]
            norm2 = norm2 + s * s
        rn = _rsqrt_nr(norm2)
        qn = norm2 * rn / ssum
        return total + qn

    total = lax.fori_loop(0, _NG, group_body, jnp.zeros((16,), jnp.float32))
    qn_v[...] = total * (1.0 / _N)
    pltpu.sync_copy(qn_v, out_hbm.at[wid])


def _sc_partial(pos):
    mesh = plsc.VectorSubcoreMesh(
        core_axis_name="c", subcore_axis_name="s",
        num_cores=_NC, num_subcores=_NS)
    f = functools.partial(
        pl.kernel,
        out_type=jax.ShapeDtypeStruct((_NW, 16), jnp.float32),
        mesh=mesh,
        compiler_params=pltpu.CompilerParams(needs_layout_passes=False),
        scratch_types=[
            pltpu.VMEM((_N,), jnp.float32),
            pltpu.VMEM((_N,), jnp.float32),
            pltpu.VMEM((_N,), jnp.float32),
            pltpu.VMEM((256,), jnp.int32),
            pltpu.VMEM((16,), jnp.float32),
        ],
    )(_sc_body)
    return f(pos[:, 0], pos[:, 1], pos[:, 2])


def _tc_block(pb_ref, pt_ref, out_ref):
    i = pl.program_id(0)
    pb = pb_ref[...]
    pt = pt_ref[...]

    ax = pb[:, 0:1]
    ay = pb[:, 1:2]
    az = pb[:, 2:3]
    dx = pt[0:1, :] - ax
    dy = pt[1:2, :] - ay
    dz = pt[2:3, :] - az
    d2 = dx * dx + dy * dy + dz * dz

    col = lax.broadcasted_iota(jnp.int32, (_ROWS, _N), 1)
    work = d2
    mask = jnp.zeros((_ROWS, _N), jnp.float32)
    for k in range(_NUM_NBS + 1):
        m = jnp.min(work, axis=1, keepdims=True)
        eq = work == m
        jmin = jnp.min(jnp.where(eq, col, _N), axis=1, keepdims=True)
        oh = col == jmin
        if k > 0:
            mask = jnp.where(oh, 1.0, mask)
        work = jnp.where(oh, jnp.inf, work)

    d = jnp.sqrt(d2)
    inv = jnp.where(d > 0.0, 1.0 / jnp.where(d > 0.0, d, 1.0), 0.0)
    ux = dx * inv
    uy = dy * inv
    uz = dz * inv

    t = d - _D0
    sig = jnp.exp(t * t * (-1.0 / (2.0 * _R0 * _R0))) * mask
    ssum = jnp.sum(sig, axis=1, keepdims=True)

    z2 = uz * uz
    p = [
        (((231.0 * z2 - 315.0) * z2 + 105.0) * z2 - 5.0) * (1.0 / 16.0),
        ((1386.0 * z2 - 1260.0) * z2 + 210.0) * uz * (1.0 / 16.0),
        ((6930.0 * z2 - 3780.0) * z2 + 210.0) * (1.0 / 16.0),
        (27720.0 * z2 - 7560.0) * uz * (1.0 / 16.0),
        (83160.0 * z2 - 7560.0) * (1.0 / 16.0),
        10395.0 * uz,
    ]

    acc = _C[0] * jnp.sum(sig * p[0], axis=1, keepdims=True)
    norm2 = acc * acc
    a = ux
    b = uy
    for m in range(1, 7):
        if m < 6:
            accp = _C[m] * jnp.sum(sig * (p[m] * a), axis=1, keepdims=True)
            accm = _C[m] * jnp.sum(sig * (p[m] * b), axis=1, keepdims=True)
            a, b = a * ux - b * uy, a * uy + b * ux
        else:
            accp = (_C[6] * 10395.0) * jnp.sum(sig * a, axis=1, keepdims=True)
            accm = (_C[6] * 10395.0) * jnp.sum(sig * b, axis=1, keepdims=True)
        norm2 = norm2 + accp * accp + accm * accm

    qn = jnp.sqrt(norm2) / ssum
    part = jnp.sum(qn, axis=0, keepdims=True) * (1.0 / _N)

    @pl.when(i == 0)
    def _():
        out_ref[...] = jnp.zeros((1, 1), jnp.float32)

    out_ref[...] += part


def _tc_partial(pos):
    pos_t = pos.T
    out = pl.pallas_call(
        _tc_block,
        grid=(_TC_N // _ROWS,),
        in_specs=[
            pl.BlockSpec((_ROWS, 3), lambda i: (i, 0)),
            pl.BlockSpec((3, _N), lambda i: (0, 0)),
        ],
        out_specs=pl.BlockSpec((1, 1), lambda i: (0, 0)),
        out_shape=jax.ShapeDtypeStruct((1, 1), jnp.float32),
    )(pos, pos_t)
    return out[0, 0]


def kernel(positions):
    pos = positions.astype(jnp.float32)
    sc_out = _sc_partial(pos)
    tc_out = _tc_partial(pos)
    return tc_out + jnp.sum(sc_out)


# speedup vs baseline: 2.5587x; 1.1235x over previous
import functools
import math

import jax
import jax.numpy as jnp
from jax import lax
from jax.experimental import pallas as pl
from jax.experimental.pallas import tpu as pltpu
from jax.experimental.pallas import tpu_sc as plsc

_NUM_NBS = 12
_R0 = 0.5
_D0 = 0.3
_N = 4096

_TC_N = 2560
_SC_N = _N - _TC_N

_NC = 2
_NS = 16
_NW = _NC * _NS
_APW = _SC_N // _NW
_NG = _APW // 16

_ROWS = 256


def _sh6_constants():
    l = 6
    n = []
    for m in range(0, l + 1):
        nlm = math.sqrt(
            (2 * l + 1) / (4.0 * math.pi)
            * math.factorial(l - m) / math.factorial(l + m))
        n.append(nlm if m == 0 else math.sqrt(2.0) * nlm)
    return n


_C = _sh6_constants()


def _rsqrt_nr(x):
    i = plsc.bitcast(x, jnp.int32)
    i = jnp.int32(0x5F3759DF) - lax.shift_right_logical(i, 1)
    y = plsc.bitcast(i, jnp.float32)
    for _ in range(3):
        y = y * (1.5 - 0.5 * x * y * y)
    return y


def _sc_body(x_hbm, y_hbm, z_hbm, out_hbm, x_v, y_v, z_v, nbr_v, qn_v):
    wid = lax.axis_index("s") * _NC + lax.axis_index("c")
    base = _TC_N + wid * _APW
    lane = jnp.arange(16, dtype=jnp.int32)

    pltpu.sync_copy(x_hbm, x_v)
    pltpu.sync_copy(y_hbm, y_v)
    pltpu.sync_copy(z_hbm, z_v)

    inf = jnp.array(jnp.inf, jnp.float32)

    def group_body(g, total):
        gbase = base + g * 16

        def atom_body(aa, _):
            gi = gbase + aa
            gi_v = jnp.full((16,), 0, jnp.int32) + gi
            ax = plsc.load_gather(x_v, [gi_v])
            ay = plsc.load_gather(y_v, [gi_v])
            az = plsc.load_gather(z_v, [gi_v])

            def chunk_body(c, carry):
                rk, rv, rmax = carry
                sl = pl.ds(c * 16, 16)
                dx = x_v[sl] - ax
                dy = y_v[sl] - ay
                dz = z_v[sl] - az
                d2 = dx * dx + dy * dy + dz * dz
                vals = lane + c * 16
                d2 = jnp.where(vals == gi_v, inf, d2)

                def merge(_):
                    ck, cv = plsc.sort_key_val(d2, vals)
                    rck = lax.rev(ck, (0,))
                    rcv = lax.rev(cv, (0,))
                    sel = rk <= rck
                    mk = jnp.where(sel, rk, rck)
                    mv = jnp.where(sel, rv, rcv)
                    nk, nv = plsc.sort_key_val(mk, mv)
                    return nk, nv, jnp.max(nk)

                def keep(_):
                    return rk, rv, rmax

                return lax.cond(jnp.min(d2) < rmax, merge, keep, 0)

            rk0 = jnp.full((16,), jnp.inf, jnp.float32)
            rv0 = jnp.zeros((16,), jnp.int32)
            _, rv, _ = lax.fori_loop(0, _N // 16, chunk_body, (rk0, rv0, inf))
            plsc.store_scatter(nbr_v, [aa * 16 + lane], rv)
            return 0

        lax.fori_loop(0, 16, atom_body, 0)

        ax16 = x_v[pl.ds(gbase, 16)]
        ay16 = y_v[pl.ds(gbase, 16)]
        az16 = z_v[pl.ds(gbase, 16)]
        zero = jnp.zeros((16,), jnp.float32)
        acc = [zero] * 13
        ssum = zero
        for k in range(_NUM_NBS):
            nidx = plsc.load_gather(nbr_v, [lane * 16 + k])
            dx = plsc.load_gather(x_v, [nidx]) - ax16
            dy = plsc.load_gather(y_v, [nidx]) - ay16
            dz = plsc.load_gather(z_v, [nidx]) - az16
            d2 = dx * dx + dy * dy + dz * dz
            r = _rsqrt_nr(d2)
            dd = d2 * r
            ux = dx * r
            uy = dy * r
            uz = dz * r
            t = dd - _D0
            sig = jnp.exp(t * t * (-1.0 / (2.0 * _R0 * _R0)))
            ssum = ssum + sig
            z2 = uz * uz
            p = [
                (((231.0 * z2 - 315.0) * z2 + 105.0) * z2 - 5.0) * (1.0 / 16.0),
                ((1386.0 * z2 - 1260.0) * z2 + 210.0) * uz * (1.0 / 16.0),
                ((6930.0 * z2 - 3780.0) * z2 + 210.0) * (1.0 / 16.0),
                (27720.0 * z2 - 7560.0) * uz * (1.0 / 16.0),
                (83160.0 * z2 - 7560.0) * (1.0 / 16.0),
                10395.0 * uz,
            ]
            acc[0] = acc[0] + sig * p[0]
            a = ux
            b = uy
            for m in range(1, 7):
                if m < 6:
                    acc[2 * m - 1] = acc[2 * m - 1] + sig * (p[m] * a)
                    acc[2 * m] = acc[2 * m] + sig * (p[m] * b)
                    a, b = a * ux - b * uy, a * uy + b * ux
                else:
                    acc[11] = acc[11] + sig * a
                    acc[12] = acc[12] + sig * b
        norm2 = jnp.zeros((16,), jnp.float32)
        cs = [_C[0]] + [c for m in range(1, 7) for c in
                        ((_C[m], _C[m]) if m < 6 else
                         (_C[6] * 10395.0, _C[6] * 10395.0))]
        for m in range(13):
            s = cs[m] * acc[m]
            norm2 = norm2 + s * s
        rn = _rsqrt_nr(norm2)
        qn = norm2 * rn / ssum
        return total + qn

    total = lax.fori_loop(0, _NG, group_body, jnp.zeros((16,), jnp.float32))
    qn_v[...] = total * (1.0 / _N)
    pltpu.sync_copy(qn_v, out_hbm.at[wid])


def _sc_partial(pos):
    mesh = plsc.VectorSubcoreMesh(
        core_axis_name="c", subcore_axis_name="s",
        num_cores=_NC, num_subcores=_NS)
    f = functools.partial(
        pl.kernel,
        out_type=jax.ShapeDtypeStruct((_NW, 16), jnp.float32),
        mesh=mesh,
        compiler_params=pltpu.CompilerParams(needs_layout_passes=False),
        scratch_types=[
            pltpu.VMEM((_N,), jnp.float32),
            pltpu.VMEM((_N,), jnp.float32),
            pltpu.VMEM((_N,), jnp.float32),
            pltpu.VMEM((256,), jnp.int32),
            pltpu.VMEM((16,), jnp.float32),
        ],
    )(_sc_body)
    return f(pos[:, 0], pos[:, 1], pos[:, 2])


def _tc_block(pb_ref, pt_ref, out_ref):
    i = pl.program_id(0)
    pb = pb_ref[...]
    pt = pt_ref[...]

    ax = pb[:, 0:1]
    ay = pb[:, 1:2]
    az = pb[:, 2:3]
    dx = pt[0:1, :] - ax
    dy = pt[1:2, :] - ay
    dz = pt[2:3, :] - az
    d2 = dx * dx + dy * dy + dz * dz

    col = lax.broadcasted_iota(jnp.int32, (_ROWS, _N), 1)
    work = d2
    zeroRN = jnp.zeros((_ROWS, _N), jnp.float32)
    d2s, dxs, dys, dzs = [], [], [], []
    for k in range(_NUM_NBS + 1):
        m = jnp.min(work, axis=1, keepdims=True)
        eq = work == m
        jmin = jnp.min(jnp.where(eq, col, _N), axis=1, keepdims=True)
        oh = col == jmin
        if k > 0:
            d2s.append(m)
            dxs.append(jnp.sum(jnp.where(oh, dx, zeroRN), axis=1,
                               keepdims=True))
            dys.append(jnp.sum(jnp.where(oh, dy, zeroRN), axis=1,
                               keepdims=True))
            dzs.append(jnp.sum(jnp.where(oh, dz, zeroRN), axis=1,
                               keepdims=True))
        work = jnp.where(oh, jnp.inf, work)

    d2n = jnp.concatenate(d2s, axis=1)
    dx = jnp.concatenate(dxs, axis=1)
    dy = jnp.concatenate(dys, axis=1)
    dz = jnp.concatenate(dzs, axis=1)

    d = jnp.sqrt(d2n)
    inv = 1.0 / d
    ux = dx * inv
    uy = dy * inv
    uz = dz * inv

    t = d - _D0
    sig = jnp.exp(t * t * (-1.0 / (2.0 * _R0 * _R0)))
    ssum = jnp.sum(sig, axis=1, keepdims=True)

    z2 = uz * uz
    p = [
        (((231.0 * z2 - 315.0) * z2 + 105.0) * z2 - 5.0) * (1.0 / 16.0),
        ((1386.0 * z2 - 1260.0) * z2 + 210.0) * uz * (1.0 / 16.0),
        ((6930.0 * z2 - 3780.0) * z2 + 210.0) * (1.0 / 16.0),
        (27720.0 * z2 - 7560.0) * uz * (1.0 / 16.0),
        (83160.0 * z2 - 7560.0) * (1.0 / 16.0),
        10395.0 * uz,
    ]

    acc = _C[0] * jnp.sum(sig * p[0], axis=1, keepdims=True)
    norm2 = acc * acc
    a = ux
    b = uy
    for m in range(1, 7):
        if m < 6:
            accp = _C[m] * jnp.sum(sig * (p[m] * a), axis=1, keepdims=True)
            accm = _C[m] * jnp.sum(sig * (p[m] * b), axis=1, keepdims=True)
            a, b = a * ux - b * uy, a * uy + b * ux
        else:
            accp = (_C[6] * 10395.0) * jnp.sum(sig * a, axis=1, keepdims=True)
            accm = (_C[6] * 10395.0) * jnp.sum(sig * b, axis=1, keepdims=True)
        norm2 = norm2 + accp * accp + accm * accm

    qn = jnp.sqrt(norm2) / ssum
    part = jnp.sum(qn, axis=0, keepdims=True) * (1.0 / _N)

    @pl.when(i == 0)
    def _():
        out_ref[...] = jnp.zeros((1, 1), jnp.float32)

    out_ref[...] += part


def _tc_partial(pos):
    pos_t = pos.T
    out = pl.pallas_call(
        _tc_block,
        grid=(_TC_N // _ROWS,),
        in_specs=[
            pl.BlockSpec((_ROWS, 3), lambda i: (i, 0)),
            pl.BlockSpec((3, _N), lambda i: (0, 0)),
        ],
        out_specs=pl.BlockSpec((1, 1), lambda i: (0, 0)),
        out_shape=jax.ShapeDtypeStruct((1, 1), jnp.float32),
    )(pos, pos_t)
    return out[0, 0]


def kernel(positions):
    pos = positions.astype(jnp.float32)
    sc_out = _sc_partial(pos)
    tc_out = _tc_partial(pos)
    return tc_out + jnp.sum(sc_out)


# speedup vs baseline: 2.9629x; 1.1580x over previous
import functools
import math

import jax
import jax.numpy as jnp
from jax import lax
from jax.experimental import pallas as pl
from jax.experimental.pallas import tpu as pltpu
from jax.experimental.pallas import tpu_sc as plsc

_NUM_NBS = 12
_R0 = 0.5
_D0 = 0.3
_N = 4096

_TC_N = 3072
_SC_N = _N - _TC_N

_NC = 2
_NS = 16
_NW = _NC * _NS
_APW = _SC_N // _NW
_NG = _APW // 16

_ROWS = 256


def _sh6_constants():
    l = 6
    n = []
    for m in range(0, l + 1):
        nlm = math.sqrt(
            (2 * l + 1) / (4.0 * math.pi)
            * math.factorial(l - m) / math.factorial(l + m))
        n.append(nlm if m == 0 else math.sqrt(2.0) * nlm)
    return n


_C = _sh6_constants()


def _rsqrt_nr(x):
    i = plsc.bitcast(x, jnp.int32)
    i = jnp.int32(0x5F3759DF) - lax.shift_right_logical(i, 1)
    y = plsc.bitcast(i, jnp.float32)
    for _ in range(3):
        y = y * (1.5 - 0.5 * x * y * y)
    return y


def _sc_body(x_hbm, y_hbm, z_hbm, out_hbm, x_v, y_v, z_v, nbr_v, qn_v):
    wid = lax.axis_index("s") * _NC + lax.axis_index("c")
    base = _TC_N + wid * _APW
    lane = jnp.arange(16, dtype=jnp.int32)

    pltpu.sync_copy(x_hbm, x_v)
    pltpu.sync_copy(y_hbm, y_v)
    pltpu.sync_copy(z_hbm, z_v)

    inf = jnp.array(jnp.inf, jnp.float32)

    def group_body(g, total):
        gbase = base + g * 16

        def atom_body(aa, _):
            gi = gbase + aa
            gi_v = jnp.full((16,), 0, jnp.int32) + gi
            ax = plsc.load_gather(x_v, [gi_v])
            ay = plsc.load_gather(y_v, [gi_v])
            az = plsc.load_gather(z_v, [gi_v])

            def chunk_body(c, carry):
                rk, rv, rmax = carry
                sl = pl.ds(c * 16, 16)
                dx = x_v[sl] - ax
                dy = y_v[sl] - ay
                dz = z_v[sl] - az
                d2 = dx * dx + dy * dy + dz * dz
                vals = lane + c * 16
                d2 = jnp.where(vals == gi_v, inf, d2)

                def merge(_):
                    ck, cv = plsc.sort_key_val(d2, vals)
                    rck = lax.rev(ck, (0,))
                    rcv = lax.rev(cv, (0,))
                    sel = rk <= rck
                    mk = jnp.where(sel, rk, rck)
                    mv = jnp.where(sel, rv, rcv)
                    nk, nv = plsc.sort_key_val(mk, mv)
                    return nk, nv, jnp.max(nk)

                def keep(_):
                    return rk, rv, rmax

                return lax.cond(jnp.min(d2) < rmax, merge, keep, 0)

            rk0 = jnp.full((16,), jnp.inf, jnp.float32)
            rv0 = jnp.zeros((16,), jnp.int32)
            _, rv, _ = lax.fori_loop(0, _N // 16, chunk_body, (rk0, rv0, inf))
            plsc.store_scatter(nbr_v, [aa * 16 + lane], rv)
            return 0

        lax.fori_loop(0, 16, atom_body, 0)

        ax16 = x_v[pl.ds(gbase, 16)]
        ay16 = y_v[pl.ds(gbase, 16)]
        az16 = z_v[pl.ds(gbase, 16)]
        zero = jnp.zeros((16,), jnp.float32)
        acc = [zero] * 13
        ssum = zero
        for k in range(_NUM_NBS):
            nidx = plsc.load_gather(nbr_v, [lane * 16 + k])
            dx = plsc.load_gather(x_v, [nidx]) - ax16
            dy = plsc.load_gather(y_v, [nidx]) - ay16
            dz = plsc.load_gather(z_v, [nidx]) - az16
            d2 = dx * dx + dy * dy + dz * dz
            r = _rsqrt_nr(d2)
            dd = d2 * r
            ux = dx * r
            uy = dy * r
            uz = dz * r
            t = dd - _D0
            sig = jnp.exp(t * t * (-1.0 / (2.0 * _R0 * _R0)))
            ssum = ssum + sig
            z2 = uz * uz
            p = [
                (((231.0 * z2 - 315.0) * z2 + 105.0) * z2 - 5.0) * (1.0 / 16.0),
                ((1386.0 * z2 - 1260.0) * z2 + 210.0) * uz * (1.0 / 16.0),
                ((6930.0 * z2 - 3780.0) * z2 + 210.0) * (1.0 / 16.0),
                (27720.0 * z2 - 7560.0) * uz * (1.0 / 16.0),
                (83160.0 * z2 - 7560.0) * (1.0 / 16.0),
                10395.0 * uz,
            ]
            acc[0] = acc[0] + sig * p[0]
            a = ux
            b = uy
            for m in range(1, 7):
                if m < 6:
                    acc[2 * m - 1] = acc[2 * m - 1] + sig * (p[m] * a)
                    acc[2 * m] = acc[2 * m] + sig * (p[m] * b)
                    a, b = a * ux - b * uy, a * uy + b * ux
                else:
                    acc[11] = acc[11] + sig * a
                    acc[12] = acc[12] + sig * b
        norm2 = jnp.zeros((16,), jnp.float32)
        cs = [_C[0]] + [c for m in range(1, 7) for c in
                        ((_C[m], _C[m]) if m < 6 else
                         (_C[6] * 10395.0, _C[6] * 10395.0))]
        for m in range(13):
            s = cs[m] * acc[m]
            norm2 = norm2 + s * s
        rn = _rsqrt_nr(norm2)
        qn = norm2 * rn / ssum
        return total + qn

    total = lax.fori_loop(0, _NG, group_body, jnp.zeros((16,), jnp.float32))
    qn_v[...] = total * (1.0 / _N)
    pltpu.sync_copy(qn_v, out_hbm.at[wid])


def _sc_partial(pos):
    mesh = plsc.VectorSubcoreMesh(
        core_axis_name="c", subcore_axis_name="s",
        num_cores=_NC, num_subcores=_NS)
    f = functools.partial(
        pl.kernel,
        out_type=jax.ShapeDtypeStruct((_NW, 16), jnp.float32),
        mesh=mesh,
        compiler_params=pltpu.CompilerParams(needs_layout_passes=False),
        scratch_types=[
            pltpu.VMEM((_N,), jnp.float32),
            pltpu.VMEM((_N,), jnp.float32),
            pltpu.VMEM((_N,), jnp.float32),
            pltpu.VMEM((256,), jnp.int32),
            pltpu.VMEM((16,), jnp.float32),
        ],
    )(_sc_body)
    return f(pos[:, 0], pos[:, 1], pos[:, 2])


def _tc_block(pb_ref, pt_ref, out_ref):
    i = pl.program_id(0)
    pb = pb_ref[...]
    pt = pt_ref[...]

    ax = pb[:, 0:1]
    ay = pb[:, 1:2]
    az = pb[:, 2:3]
    dx = pt[0:1, :] - ax
    dy = pt[1:2, :] - ay
    dz = pt[2:3, :] - az
    d2 = dx * dx + dy * dy + dz * dz

    col = lax.broadcasted_iota(jnp.int32, (_ROWS, _N), 1)
    work = d2
    zeroRN = jnp.zeros((_ROWS, _N), jnp.float32)
    d2s, dxs, dys, dzs = [], [], [], []
    for k in range(_NUM_NBS + 1):
        m = jnp.min(work, axis=1, keepdims=True)
        eq = work == m
        jmin = jnp.min(jnp.where(eq, col, _N), axis=1, keepdims=True)
        oh = col == jmin
        if k > 0:
            d2s.append(m)
            dxs.append(jnp.sum(jnp.where(oh, dx, zeroRN), axis=1,
                               keepdims=True))
            dys.append(jnp.sum(jnp.where(oh, dy, zeroRN), axis=1,
                               keepdims=True))
            dzs.append(jnp.sum(jnp.where(oh, dz, zeroRN), axis=1,
                               keepdims=True))
        work = jnp.where(oh, jnp.inf, work)

    d2n = jnp.concatenate(d2s, axis=1)
    dx = jnp.concatenate(dxs, axis=1)
    dy = jnp.concatenate(dys, axis=1)
    dz = jnp.concatenate(dzs, axis=1)

    d = jnp.sqrt(d2n)
    inv = 1.0 / d
    ux = dx * inv
    uy = dy * inv
    uz = dz * inv

    t = d - _D0
    sig = jnp.exp(t * t * (-1.0 / (2.0 * _R0 * _R0)))
    ssum = jnp.sum(sig, axis=1, keepdims=True)

    z2 = uz * uz
    p = [
        (((231.0 * z2 - 315.0) * z2 + 105.0) * z2 - 5.0) * (1.0 / 16.0),
        ((1386.0 * z2 - 1260.0) * z2 + 210.0) * uz * (1.0 / 16.0),
        ((6930.0 * z2 - 3780.0) * z2 + 210.0) * (1.0 / 16.0),
        (27720.0 * z2 - 7560.0) * uz * (1.0 / 16.0),
        (83160.0 * z2 - 7560.0) * (1.0 / 16.0),
        10395.0 * uz,
    ]

    acc = _C[0] * jnp.sum(sig * p[0], axis=1, keepdims=True)
    norm2 = acc * acc
    a = ux
    b = uy
    for m in range(1, 7):
        if m < 6:
            accp = _C[m] * jnp.sum(sig * (p[m] * a), axis=1, keepdims=True)
            accm = _C[m] * jnp.sum(sig * (p[m] * b), axis=1, keepdims=True)
            a, b = a * ux - b * uy, a * uy + b * ux
        else:
            accp = (_C[6] * 10395.0) * jnp.sum(sig * a, axis=1, keepdims=True)
            accm = (_C[6] * 10395.0) * jnp.sum(sig * b, axis=1, keepdims=True)
        norm2 = norm2 + accp * accp + accm * accm

    qn = jnp.sqrt(norm2) / ssum
    part = jnp.sum(qn, axis=0, keepdims=True) * (1.0 / _N)

    @pl.when(i == 0)
    def _():
        out_ref[...] = jnp.zeros((1, 1), jnp.float32)

    out_ref[...] += part


def _tc_partial(pos):
    pos_t = pos.T
    out = pl.pallas_call(
        _tc_block,
        grid=(_TC_N // _ROWS,),
        in_specs=[
            pl.BlockSpec((_ROWS, 3), lambda i: (i, 0)),
            pl.BlockSpec((3, _N), lambda i: (0, 0)),
        ],
        out_specs=pl.BlockSpec((1, 1), lambda i: (0, 0)),
        out_shape=jax.ShapeDtypeStruct((1, 1), jnp.float32),
    )(pos, pos_t)
    return out[0, 0]


def kernel(positions):
    pos = positions.astype(jnp.float32)
    sc_out = _sc_partial(pos)
    tc_out = _tc_partial(pos)
    return tc_out + jnp.sum(sc_out)


# speedup vs baseline: 2.9633x; 1.0001x over previous
import functools
import math

import jax
import jax.numpy as jnp
from jax import lax
from jax.experimental import pallas as pl
from jax.experimental.pallas import tpu as pltpu
from jax.experimental.pallas import tpu_sc as plsc

_NUM_NBS = 12
_R0 = 0.5
_D0 = 0.3
_N = 4096

_TC_N = 3072
_SC_N = _N - _TC_N

_NC = 2
_NS = 16
_NW = _NC * _NS
_APW = _SC_N // _NW
_NG = _APW // 16

_ROWS = 256


def _sh6_constants():
    l = 6
    n = []
    for m in range(0, l + 1):
        nlm = math.sqrt(
            (2 * l + 1) / (4.0 * math.pi)
            * math.factorial(l - m) / math.factorial(l + m))
        n.append(nlm if m == 0 else math.sqrt(2.0) * nlm)
    return n


_C = _sh6_constants()


def _rsqrt_nr(x):
    i = plsc.bitcast(x, jnp.int32)
    i = jnp.int32(0x5F3759DF) - lax.shift_right_logical(i, 1)
    y = plsc.bitcast(i, jnp.float32)
    for _ in range(3):
        y = y * (1.5 - 0.5 * x * y * y)
    return y


def _sc_body(x_hbm, y_hbm, z_hbm, out_hbm, x_v, y_v, z_v, nbr_v, qn_v):
    wid = lax.axis_index("s") * _NC + lax.axis_index("c")
    base = _TC_N + wid * _APW
    lane = jnp.arange(16, dtype=jnp.int32)

    pltpu.sync_copy(x_hbm, x_v)
    pltpu.sync_copy(y_hbm, y_v)
    pltpu.sync_copy(z_hbm, z_v)

    inf = jnp.array(jnp.inf, jnp.float32)

    def group_body(g, total):
        gbase = base + g * 16

        def atom_body(aa, _):
            gi = gbase + aa
            gi_v = jnp.full((16,), 0, jnp.int32) + gi
            ax = plsc.load_gather(x_v, [gi_v])
            ay = plsc.load_gather(y_v, [gi_v])
            az = plsc.load_gather(z_v, [gi_v])

            def chunk_body(c4, carry):
                ds = []
                for t in range(4):
                    c = c4 * 4 + t
                    sl = pl.ds(c * 16, 16)
                    dx = x_v[sl] - ax
                    dy = y_v[sl] - ay
                    dz = z_v[sl] - az
                    d2 = dx * dx + dy * dy + dz * dz
                    vals = lane + c * 16
                    d2 = jnp.where(vals == gi_v, inf, d2)
                    ds.append((d2, vals))
                m4 = jnp.minimum(jnp.minimum(ds[0][0], ds[1][0]),
                                 jnp.minimum(ds[2][0], ds[3][0]))

                def merge_all(cc):
                    for t in range(4):
                        d2, vals = ds[t]

                        def do_merge(cc2, d2=d2, vals=vals):
                            rk, rv, rmax = cc2
                            ck, cv = plsc.sort_key_val(d2, vals)
                            rck = lax.rev(ck, (0,))
                            rcv = lax.rev(cv, (0,))
                            sel = rk <= rck
                            mk = jnp.where(sel, rk, rck)
                            mv = jnp.where(sel, rv, rcv)
                            nk, nv = plsc.sort_key_val(mk, mv)
                            return nk, nv, jnp.max(nk)

                        cc = lax.cond(jnp.min(d2) < cc[2], do_merge,
                                      lambda cc2: cc2, cc)
                    return cc

                return lax.cond(jnp.min(m4) < carry[2], merge_all,
                                lambda cc: cc, carry)

            rk0 = jnp.full((16,), jnp.inf, jnp.float32)
            rv0 = jnp.zeros((16,), jnp.int32)
            _, rv, _ = lax.fori_loop(0, _N // 64, chunk_body, (rk0, rv0, inf))
            plsc.store_scatter(nbr_v, [aa * 16 + lane], rv)
            return 0

        lax.fori_loop(0, 16, atom_body, 0)

        ax16 = x_v[pl.ds(gbase, 16)]
        ay16 = y_v[pl.ds(gbase, 16)]
        az16 = z_v[pl.ds(gbase, 16)]
        zero = jnp.zeros((16,), jnp.float32)
        acc = [zero] * 13
        ssum = zero
        for k in range(_NUM_NBS):
            nidx = plsc.load_gather(nbr_v, [lane * 16 + k])
            dx = plsc.load_gather(x_v, [nidx]) - ax16
            dy = plsc.load_gather(y_v, [nidx]) - ay16
            dz = plsc.load_gather(z_v, [nidx]) - az16
            d2 = dx * dx + dy * dy + dz * dz
            r = _rsqrt_nr(d2)
            dd = d2 * r
            ux = dx * r
            uy = dy * r
            uz = dz * r
            t = dd - _D0
            sig = jnp.exp(t * t * (-1.0 / (2.0 * _R0 * _R0)))
            ssum = ssum + sig
            z2 = uz * uz
            p = [
                (((231.0 * z2 - 315.0) * z2 + 105.0) * z2 - 5.0) * (1.0 / 16.0),
                ((1386.0 * z2 - 1260.0) * z2 + 210.0) * uz * (1.0 / 16.0),
                ((6930.0 * z2 - 3780.0) * z2 + 210.0) * (1.0 / 16.0),
                (27720.0 * z2 - 7560.0) * uz * (1.0 / 16.0),
                (83160.0 * z2 - 7560.0) * (1.0 / 16.0),
                10395.0 * uz,
            ]
            acc[0] = acc[0] + sig * p[0]
            a = ux
            b = uy
            for m in range(1, 7):
                if m < 6:
                    acc[2 * m - 1] = acc[2 * m - 1] + sig * (p[m] * a)
                    acc[2 * m] = acc[2 * m] + sig * (p[m] * b)
                    a, b = a * ux - b * uy, a * uy + b * ux
                else:
                    acc[11] = acc[11] + sig * a
                    acc[12] = acc[12] + sig * b
        norm2 = jnp.zeros((16,), jnp.float32)
        cs = [_C[0]] + [c for m in range(1, 7) for c in
                        ((_C[m], _C[m]) if m < 6 else
                         (_C[6] * 10395.0, _C[6] * 10395.0))]
        for m in range(13):
            s = cs[m] * acc[m]
            norm2 = norm2 + s * s
        rn = _rsqrt_nr(norm2)
        qn = norm2 * rn / ssum
        return total + qn

    total = lax.fori_loop(0, _NG, group_body, jnp.zeros((16,), jnp.float32))
    qn_v[...] = total * (1.0 / _N)
    pltpu.sync_copy(qn_v, out_hbm.at[wid])


def _sc_partial(pos):
    mesh = plsc.VectorSubcoreMesh(
        core_axis_name="c", subcore_axis_name="s",
        num_cores=_NC, num_subcores=_NS)
    f = functools.partial(
        pl.kernel,
        out_type=jax.ShapeDtypeStruct((_NW, 16), jnp.float32),
        mesh=mesh,
        compiler_params=pltpu.CompilerParams(needs_layout_passes=False),
        scratch_types=[
            pltpu.VMEM((_N,), jnp.float32),
            pltpu.VMEM((_N,), jnp.float32),
            pltpu.VMEM((_N,), jnp.float32),
            pltpu.VMEM((256,), jnp.int32),
            pltpu.VMEM((16,), jnp.float32),
        ],
    )(_sc_body)
    return f(pos[:, 0], pos[:, 1], pos[:, 2])


def _tc_block(pb_ref, pt_ref, out_ref):
    i = pl.program_id(0)
    pb = pb_ref[...]
    pt = pt_ref[...]

    ax = pb[:, 0:1]
    ay = pb[:, 1:2]
    az = pb[:, 2:3]
    dx = pt[0:1, :] - ax
    dy = pt[1:2, :] - ay
    dz = pt[2:3, :] - az
    d2 = dx * dx + dy * dy + dz * dz

    col = lax.broadcasted_iota(jnp.int32, (_ROWS, _N), 1)
    work = d2
    zeroRN = jnp.zeros((_ROWS, _N), jnp.float32)
    d2s, dxs, dys, dzs = [], [], [], []
    for k in range(_NUM_NBS + 1):
        m = jnp.min(work, axis=1, keepdims=True)
        eq = work == m
        jmin = jnp.min(jnp.where(eq, col, _N), axis=1, keepdims=True)
        oh = col == jmin
        if k > 0:
            d2s.append(m)
            dxs.append(jnp.sum(jnp.where(oh, dx, zeroRN), axis=1,
                               keepdims=True))
            dys.append(jnp.sum(jnp.where(oh, dy, zeroRN), axis=1,
                               keepdims=True))
            dzs.append(jnp.sum(jnp.where(oh, dz, zeroRN), axis=1,
                               keepdims=True))
        work = jnp.where(oh, jnp.inf, work)

    d2n = jnp.concatenate(d2s, axis=1)
    dx = jnp.concatenate(dxs, axis=1)
    dy = jnp.concatenate(dys, axis=1)
    dz = jnp.concatenate(dzs, axis=1)

    d = jnp.sqrt(d2n)
    inv = 1.0 / d
    ux = dx * inv
    uy = dy * inv
    uz = dz * inv

    t = d - _D0
    sig = jnp.exp(t * t * (-1.0 / (2.0 * _R0 * _R0)))
    ssum = jnp.sum(sig, axis=1, keepdims=True)

    z2 = uz * uz
    p = [
        (((231.0 * z2 - 315.0) * z2 + 105.0) * z2 - 5.0) * (1.0 / 16.0),
        ((1386.0 * z2 - 1260.0) * z2 + 210.0) * uz * (1.0 / 16.0),
        ((6930.0 * z2 - 3780.0) * z2 + 210.0) * (1.0 / 16.0),
        (27720.0 * z2 - 7560.0) * uz * (1.0 / 16.0),
        (83160.0 * z2 - 7560.0) * (1.0 / 16.0),
        10395.0 * uz,
    ]

    acc = _C[0] * jnp.sum(sig * p[0], axis=1, keepdims=True)
    norm2 = acc * acc
    a = ux
    b = uy
    for m in range(1, 7):
        if m < 6:
            accp = _C[m] * jnp.sum(sig * (p[m] * a), axis=1, keepdims=True)
            accm = _C[m] * jnp.sum(sig * (p[m] * b), axis=1, keepdims=True)
            a, b = a * ux - b * uy, a * uy + b * ux
        else:
            accp = (_C[6] * 10395.0) * jnp.sum(sig * a, axis=1, keepdims=True)
            accm = (_C[6] * 10395.0) * jnp.sum(sig * b, axis=1, keepdims=True)
        norm2 = norm2 + accp * accp + accm * accm

    qn = jnp.sqrt(norm2) / ssum
    part = jnp.sum(qn, axis=0, keepdims=True) * (1.0 / _N)

    @pl.when(i == 0)
    def _():
        out_ref[...] = jnp.zeros((1, 1), jnp.float32)

    out_ref[...] += part


def _tc_partial(pos):
    pos_t = pos.T
    out = pl.pallas_call(
        _tc_block,
        grid=(_TC_N // _ROWS,),
        in_specs=[
            pl.BlockSpec((_ROWS, 3), lambda i: (i, 0)),
            pl.BlockSpec((3, _N), lambda i: (0, 0)),
        ],
        out_specs=pl.BlockSpec((1, 1), lambda i: (0, 0)),
        out_shape=jax.ShapeDtypeStruct((1, 1), jnp.float32),
    )(pos, pos_t)
    return out[0, 0]


def kernel(positions):
    pos = positions.astype(jnp.float32)
    sc_out = _sc_partial(pos)
    tc_out = _tc_partial(pos)
    return tc_out + jnp.sum(sc_out)


# speedup vs baseline: 4.4838x; 1.5131x over previous
import functools
import math

import jax
import jax.numpy as jnp
from jax import lax
from jax.experimental import pallas as pl
from jax.experimental.pallas import tpu as pltpu
from jax.experimental.pallas import tpu_sc as plsc

_NUM_NBS = 12
_R0 = 0.5
_D0 = 0.3
_N = 4096

_TC_N = 3072
_SC_N = _N - _TC_N

_NC = 2
_NS = 16
_NW = _NC * _NS
_APW = _SC_N // _NW
_NG = _APW // 16

_ROWS = 256


def _sh6_constants():
    l = 6
    n = []
    for m in range(0, l + 1):
        nlm = math.sqrt(
            (2 * l + 1) / (4.0 * math.pi)
            * math.factorial(l - m) / math.factorial(l + m))
        n.append(nlm if m == 0 else math.sqrt(2.0) * nlm)
    return n


_C = _sh6_constants()


def _rsqrt_nr(x):
    i = plsc.bitcast(x, jnp.int32)
    i = jnp.int32(0x5F3759DF) - lax.shift_right_logical(i, 1)
    y = plsc.bitcast(i, jnp.float32)
    for _ in range(3):
        y = y * (1.5 - 0.5 * x * y * y)
    return y


def _sc_body(x_hbm, y_hbm, z_hbm, out_hbm, x_v, y_v, z_v, nbr_v, qn_v):
    wid = lax.axis_index("s") * _NC + lax.axis_index("c")
    base = _TC_N + wid * _APW
    lane = jnp.arange(16, dtype=jnp.int32)

    pltpu.sync_copy(x_hbm, x_v)
    pltpu.sync_copy(y_hbm, y_v)
    pltpu.sync_copy(z_hbm, z_v)

    inf = jnp.array(jnp.inf, jnp.float32)

    def group_body(g, total):
        gbase = base + g * 16

        def atom_body(aa, _):
            gi = gbase + aa
            gi_v = jnp.full((16,), 0, jnp.int32) + gi
            ax = plsc.load_gather(x_v, [gi_v])
            ay = plsc.load_gather(y_v, [gi_v])
            az = plsc.load_gather(z_v, [gi_v])

            def chunk_body(c4, carry):
                ds = []
                for t in range(4):
                    c = c4 * 4 + t
                    sl = pl.ds(c * 16, 16)
                    dx = x_v[sl] - ax
                    dy = y_v[sl] - ay
                    dz = z_v[sl] - az
                    d2 = dx * dx + dy * dy + dz * dz
                    vals = lane + c * 16
                    d2 = jnp.where(vals == gi_v, inf, d2)
                    ds.append((d2, vals))
                m4 = jnp.minimum(jnp.minimum(ds[0][0], ds[1][0]),
                                 jnp.minimum(ds[2][0], ds[3][0]))

                def merge_all(cc):
                    for t in range(4):
                        d2, vals = ds[t]

                        def do_merge(cc2, d2=d2, vals=vals):
                            rk, rv, rmax = cc2
                            ck, cv = plsc.sort_key_val(d2, vals)
                            rck = lax.rev(ck, (0,))
                            rcv = lax.rev(cv, (0,))
                            sel = rk <= rck
                            mk = jnp.where(sel, rk, rck)
                            mv = jnp.where(sel, rv, rcv)
                            nk, nv = plsc.sort_key_val(mk, mv)
                            return nk, nv, jnp.max(nk)

                        cc = lax.cond(jnp.min(d2) < cc[2], do_merge,
                                      lambda cc2: cc2, cc)
                    return cc

                return lax.cond(jnp.min(m4) < carry[2], merge_all,
                                lambda cc: cc, carry)

            rk0 = jnp.full((16,), jnp.inf, jnp.float32)
            rv0 = jnp.zeros((16,), jnp.int32)
            _, rv, _ = lax.fori_loop(0, _N // 64, chunk_body, (rk0, rv0, inf))
            plsc.store_scatter(nbr_v, [aa * 16 + lane], rv)
            return 0

        lax.fori_loop(0, 16, atom_body, 0)

        ax16 = x_v[pl.ds(gbase, 16)]
        ay16 = y_v[pl.ds(gbase, 16)]
        az16 = z_v[pl.ds(gbase, 16)]
        zero = jnp.zeros((16,), jnp.float32)
        acc = [zero] * 13
        ssum = zero
        for k in range(_NUM_NBS):
            nidx = plsc.load_gather(nbr_v, [lane * 16 + k])
            dx = plsc.load_gather(x_v, [nidx]) - ax16
            dy = plsc.load_gather(y_v, [nidx]) - ay16
            dz = plsc.load_gather(z_v, [nidx]) - az16
            d2 = dx * dx + dy * dy + dz * dz
            r = _rsqrt_nr(d2)
            dd = d2 * r
            ux = dx * r
            uy = dy * r
            uz = dz * r
            t = dd - _D0
            sig = jnp.exp(t * t * (-1.0 / (2.0 * _R0 * _R0)))
            ssum = ssum + sig
            z2 = uz * uz
            p = [
                (((231.0 * z2 - 315.0) * z2 + 105.0) * z2 - 5.0) * (1.0 / 16.0),
                ((1386.0 * z2 - 1260.0) * z2 + 210.0) * uz * (1.0 / 16.0),
                ((6930.0 * z2 - 3780.0) * z2 + 210.0) * (1.0 / 16.0),
                (27720.0 * z2 - 7560.0) * uz * (1.0 / 16.0),
                (83160.0 * z2 - 7560.0) * (1.0 / 16.0),
                10395.0 * uz,
            ]
            acc[0] = acc[0] + sig * p[0]
            a = ux
            b = uy
            for m in range(1, 7):
                if m < 6:
                    acc[2 * m - 1] = acc[2 * m - 1] + sig * (p[m] * a)
                    acc[2 * m] = acc[2 * m] + sig * (p[m] * b)
                    a, b = a * ux - b * uy, a * uy + b * ux
                else:
                    acc[11] = acc[11] + sig * a
                    acc[12] = acc[12] + sig * b
        norm2 = jnp.zeros((16,), jnp.float32)
        cs = [_C[0]] + [c for m in range(1, 7) for c in
                        ((_C[m], _C[m]) if m < 6 else
                         (_C[6] * 10395.0, _C[6] * 10395.0))]
        for m in range(13):
            s = cs[m] * acc[m]
            norm2 = norm2 + s * s
        rn = _rsqrt_nr(norm2)
        qn = norm2 * rn / ssum
        return total + qn

    total = lax.fori_loop(0, _NG, group_body, jnp.zeros((16,), jnp.float32))
    qn_v[...] = total * (1.0 / _N)
    pltpu.sync_copy(qn_v, out_hbm.at[wid])


def _sc_partial(pos):
    mesh = plsc.VectorSubcoreMesh(
        core_axis_name="c", subcore_axis_name="s",
        num_cores=_NC, num_subcores=_NS)
    f = functools.partial(
        pl.kernel,
        out_type=jax.ShapeDtypeStruct((_NW, 16), jnp.float32),
        mesh=mesh,
        compiler_params=pltpu.CompilerParams(needs_layout_passes=False),
        scratch_types=[
            pltpu.VMEM((_N,), jnp.float32),
            pltpu.VMEM((_N,), jnp.float32),
            pltpu.VMEM((_N,), jnp.float32),
            pltpu.VMEM((256,), jnp.int32),
            pltpu.VMEM((16,), jnp.float32),
        ],
    )(_sc_body)
    return f(pos[:, 0], pos[:, 1], pos[:, 2])


def _tc_block(pb_ref, pt_ref, pf_ref, out_ref):
    i = pl.program_id(0)
    pb = pb_ref[...]
    pt = pt_ref[...]
    pf = pf_ref[...]

    ax = pb[:, 0:1]
    ay = pb[:, 1:2]
    az = pb[:, 2:3]
    dx = pt[0:1, :] - ax
    dy = pt[1:2, :] - ay
    dz = pt[2:3, :] - az
    d2 = dx * dx + dy * dy + dz * dz

    col = lax.broadcasted_iota(jnp.int32, (_ROWS, _N), 1)
    work = d2
    d2s, nbrs = [], []
    for k in range(_NUM_NBS + 1):
        m = jnp.min(work, axis=1, keepdims=True)
        eq = work == m
        jmin = jnp.min(jnp.where(eq, col, _N), axis=1, keepdims=True)
        oh = col == jmin
        if k > 0:
            d2s.append(m)
            nbrs.append(jnp.dot(oh.astype(jnp.float32), pf,
                                preferred_element_type=jnp.float32))
        work = jnp.where(oh, jnp.inf, work)

    d2n = jnp.concatenate(d2s, axis=1)
    dx = jnp.concatenate([nb[:, 0:1] for nb in nbrs], axis=1) - ax
    dy = jnp.concatenate([nb[:, 1:2] for nb in nbrs], axis=1) - ay
    dz = jnp.concatenate([nb[:, 2:3] for nb in nbrs], axis=1) - az

    d = jnp.sqrt(d2n)
    inv = 1.0 / d
    ux = dx * inv
    uy = dy * inv
    uz = dz * inv

    t = d - _D0
    sig = jnp.exp(t * t * (-1.0 / (2.0 * _R0 * _R0)))
    ssum = jnp.sum(sig, axis=1, keepdims=True)

    z2 = uz * uz
    p = [
        (((231.0 * z2 - 315.0) * z2 + 105.0) * z2 - 5.0) * (1.0 / 16.0),
        ((1386.0 * z2 - 1260.0) * z2 + 210.0) * uz * (1.0 / 16.0),
        ((6930.0 * z2 - 3780.0) * z2 + 210.0) * (1.0 / 16.0),
        (27720.0 * z2 - 7560.0) * uz * (1.0 / 16.0),
        (83160.0 * z2 - 7560.0) * (1.0 / 16.0),
        10395.0 * uz,
    ]

    acc = _C[0] * jnp.sum(sig * p[0], axis=1, keepdims=True)
    norm2 = acc * acc
    a = ux
    b = uy
    for m in range(1, 7):
        if m < 6:
            accp = _C[m] * jnp.sum(sig * (p[m] * a), axis=1, keepdims=True)
            accm = _C[m] * jnp.sum(sig * (p[m] * b), axis=1, keepdims=True)
            a, b = a * ux - b * uy, a * uy + b * ux
        else:
            accp = (_C[6] * 10395.0) * jnp.sum(sig * a, axis=1, keepdims=True)
            accm = (_C[6] * 10395.0) * jnp.sum(sig * b, axis=1, keepdims=True)
        norm2 = norm2 + accp * accp + accm * accm

    qn = jnp.sqrt(norm2) / ssum
    part = jnp.sum(qn, axis=0, keepdims=True) * (1.0 / _N)

    @pl.when(i == 0)
    def _():
        out_ref[...] = jnp.zeros((1, 1), jnp.float32)

    out_ref[...] += part


def _tc_partial(pos):
    pos_t = pos.T
    out = pl.pallas_call(
        _tc_block,
        grid=(_TC_N // _ROWS,),
        in_specs=[
            pl.BlockSpec((_ROWS, 3), lambda i: (i, 0)),
            pl.BlockSpec((3, _N), lambda i: (0, 0)),
            pl.BlockSpec((_N, 3), lambda i: (0, 0)),
        ],
        out_specs=pl.BlockSpec((1, 1), lambda i: (0, 0)),
        out_shape=jax.ShapeDtypeStruct((1, 1), jnp.float32),
    )(pos, pos_t, pos)
    return out[0, 0]


def kernel(positions):
    pos = positions.astype(jnp.float32)
    sc_out = _sc_partial(pos)
    tc_out = _tc_partial(pos)
    return tc_out + jnp.sum(sc_out)
